# baseline (speedup 1.0000x reference)
import functools

import jax
import jax.numpy as jnp
from jax import lax
from jax.experimental import pallas as pl
from jax.experimental.pallas import tpu as pltpu
from jax.experimental.pallas import tpu_sc as plsc

NUM_CORES = 2
NUM_SUBCORES = 16
NUM_WORKERS = NUM_CORES * NUM_SUBCORES
LANES = 16
BATCH_N = 16384
FEAT = 64
PAIR = 2 * FEAT
NUM_ROWS = 1000000
HALF = NUM_ROWS // 2
ROWS_PER_W = BATCH_N // NUM_WORKERS
CHUNK = 128
NCHUNK = ROWS_PER_W // CHUNK


def _body(user_hbm, item_hbm, upair_hbm, iemb_hbm, out_hbm,
          uidx_v, iidx_v, upidx_v, urows_v, irows_v, out_v, sem):
    wid = lax.axis_index("s") * NUM_CORES + lax.axis_index("c")
    base = wid * ROWS_PER_W

    pltpu.sync_copy(user_hbm.at[pl.ds(base, ROWS_PER_W)], uidx_v)
    pltpu.sync_copy(item_hbm.at[pl.ds(base, ROWS_PER_W)], iidx_v)

    def pair_body(g, _):
        sl = pl.ds(g * LANES, LANES)
        uv = uidx_v[sl]
        upidx_v[sl] = uv - jnp.where(uv >= HALF, HALF, 0)
        return ()

    lax.fori_loop(0, ROWS_PER_W // LANES, pair_body, ())

    copies = []
    for j in range(NCHUNK):
        sl = pl.ds(j * CHUNK, CHUNK)
        copies.append(pltpu.async_copy(
            upair_hbm.at[upidx_v.at[sl]], urows_v.at[sl], sem))
        copies.append(pltpu.async_copy(
            iemb_hbm.at[iidx_v.at[sl]], irows_v.at[sl], sem))
    for c in copies:
        c.wait()

    lane = lax.iota(jnp.int32, LANES)

    def blk_body(blk, _):
        acc16 = jnp.zeros((LANES,), jnp.float32)
        uvec = uidx_v[pl.ds(blk * LANES, LANES)]
        for rr in range(LANES):
            k = blk * LANES + rr
            ubase = jnp.where(uvec[rr] >= HALF, FEAT, 0)
            parts = []
            for j in range(FEAT // LANES):
                u = urows_v[k, pl.ds(ubase + j * LANES, LANES)]
                i = irows_v[k, pl.ds(j * LANES, LANES)]
                parts.append(u * i)
            s = (parts[0] + parts[1]) + (parts[2] + parts[3])
            tot = jnp.sum(s)
            acc16 = jnp.where(lane == rr, tot, acc16)
        out_v[pl.ds(blk * LANES, LANES)] = acc16
        return ()

    lax.fori_loop(0, ROWS_PER_W // LANES, blk_body, ())

    pltpu.sync_copy(out_v, out_hbm.at[pl.ds(base, ROWS_PER_W)])


def _repack_body(a_ref, o_ref):
    h = pl.program_id(1)

    @pl.when(h == 0)
    def _():
        o_ref[:, 0:FEAT] = a_ref[...]

    @pl.when(h == 1)
    def _():
        o_ref[:, FEAT:PAIR] = a_ref[...]


def _tc_repack(emb):
    nblk = 100
    blk = HALF // nblk
    return pl.pallas_call(
        _repack_body,
        out_shape=jax.ShapeDtypeStruct((HALF, PAIR), jnp.float32),
        grid=(nblk, 2),
        in_specs=[
            pl.BlockSpec((blk, FEAT), lambda b, h, n=nblk: (b + h * n, 0)),
        ],
        out_specs=pl.BlockSpec((blk, PAIR), lambda b, h: (b, 0)),
    )(emb)


@jax.jit
def kernel(user, item, users_emb, items_emb):
    upair = _tc_repack(users_emb)
    mesh = plsc.VectorSubcoreMesh(core_axis_name="c", subcore_axis_name="s")
    k = pl.kernel(
        _body,
        out_type=jax.ShapeDtypeStruct((BATCH_N,), jnp.float32),
        mesh=mesh,
        scratch_types=[
            pltpu.VMEM((ROWS_PER_W,), jnp.int32),
            pltpu.VMEM((ROWS_PER_W,), jnp.int32),
            pltpu.VMEM((ROWS_PER_W,), jnp.int32),
            pltpu.VMEM((ROWS_PER_W, PAIR), jnp.float32),
            pltpu.VMEM((ROWS_PER_W, FEAT), jnp.float32),
            pltpu.VMEM((ROWS_PER_W,), jnp.float32),
            pltpu.SemaphoreType.DMA,
        ],
        compiler_params=pltpu.CompilerParams(
            needs_layout_passes=False, use_tc_tiling_on_sc=False),
    )
    return k(user.astype(jnp.int32), item.astype(jnp.int32),
             upair, items_emb)
